# Initial kernel scaffold; baseline (speedup 1.0000x reference)
import jax
import jax.numpy as jnp
from jax.experimental import pallas as pl


def kernel(x, edge_index, Wl0, bl0, Wr0, lnw0, lnb0, a0, Wskip, Wl1, bl1, Wr1, lnw1, lnb1, a1):
    raise NotImplementedError("write your pallas kernel here")


# speedup vs baseline: 3.7060x; 3.7060x over previous
import jax
import jax.numpy as jnp
from jax import lax
from jax.experimental import pallas as pl
from jax.experimental.pallas import tpu as pltpu
from jax.experimental.pallas import tpu_sc as plsc

N = 10000
D = 128
E = 320000
EPS = 1e-5

NC = 2
NS = 16
NW = NC * NS
CHUNK = 128
NCH = 80
EPW = NCH * CHUNK
EPAD = NW * EPW
NPAD = 10112
RPT = NPAD // NS

_MESH = plsc.VectorSubcoreMesh(core_axis_name="c", subcore_axis_name="s",
                               num_cores=NC, num_subcores=NS)


def _sc_agg_body(h_hbm, srcp, dstp, znd, agg_out, src_v, dst_v, rows, agg_sh):
    c = lax.axis_index("c")
    s = lax.axis_index("s")
    wid = c * NS + s
    row0 = pl.multiple_of(s * RPT, 8)

    pltpu.sync_copy(znd.at[pl.ds(row0, RPT)], agg_sh.at[pl.ds(row0, RPT)])
    pltpu.sync_copy(srcp.at[wid], src_v)
    pltpu.sync_copy(dstp.at[wid], dst_v)
    plsc.subcore_barrier()

    def step(j, carry):
        pltpu.sync_copy(h_hbm.at[src_v.at[j]], rows)
        pltpu.sync_copy(rows, agg_sh.at[dst_v.at[j]], add=True)
        return carry

    lax.fori_loop(0, NCH, step, 0)
    plsc.subcore_barrier()
    pltpu.sync_copy(agg_sh.at[pl.ds(row0, RPT)],
                    agg_out.at[c, pl.ds(row0, RPT)])


_sc_agg = pl.kernel(
    _sc_agg_body,
    out_type=[jax.ShapeDtypeStruct((NC, NPAD, D), jnp.float32)],
    mesh=_MESH,
    scratch_types=[
        pltpu.VMEM((NCH, CHUNK), jnp.int32),
        pltpu.VMEM((NCH, CHUNK), jnp.int32),
        pltpu.VMEM((CHUNK, D), jnp.float32),
        pltpu.VMEM_SHARED((NPAD, D), jnp.float32),
    ],
    name="sc_segment_sum",
)


def _sc_cnt_body(dstp, cnt_out, dst_v, hist_v):
    c = lax.axis_index("c")
    s = lax.axis_index("s")
    wid = c * NS + s
    pltpu.sync_copy(dstp.at[wid], dst_v)

    def zstep(i, carry):
        hist_v[pl.ds(i * 16, 16)] = jnp.zeros((16,), jnp.float32)
        return carry

    lax.fori_loop(0, NPAD // 16, zstep, 0)
    ones16 = jnp.ones((16,), jnp.float32)

    def step(j, carry):
        def sub(k, carry2):
            idx = dst_v[j, pl.ds(k * 16, 16)]
            plsc.addupdate_scatter(hist_v, [idx], ones16)
            return carry2
        return lax.fori_loop(0, CHUNK // 16, sub, carry)

    lax.fori_loop(0, NCH, step, 0)
    pltpu.sync_copy(hist_v, cnt_out.at[wid])


_sc_cnt = pl.kernel(
    _sc_cnt_body,
    out_type=[jax.ShapeDtypeStruct((NW, NPAD), jnp.float32)],
    mesh=_MESH,
    scratch_types=[
        pltpu.VMEM((NCH, CHUNK), jnp.int32),
        pltpu.VMEM((NPAD,), jnp.float32),
    ],
    compiler_params=pltpu.CompilerParams(needs_layout_passes=False),
    name="sc_degree_histogram",
)


def _row_mask():
    rows = lax.broadcasted_iota(jnp.int32, (NPAD, 1), 0)
    return rows < N


def _dense_layer(aggp, cntp, h, Wl, bl, Wr, lnw, lnb, a):
    mask = _row_mask()
    cnt = lax.dot_general(cntp, jnp.ones((NW, 1), jnp.float32),
                          (((0,), (0,)), ((), ())),
                          preferred_element_type=jnp.float32)
    agg = (aggp[0] + aggp[1]) / jnp.maximum(cnt, 1.0)
    t = (jnp.dot(agg, Wl.T, preferred_element_type=jnp.float32) + bl
         + jnp.dot(h, Wr.T, preferred_element_type=jnp.float32))
    t = jnp.where(mask, t, 0.0)
    denom = float(N * D)
    mu = jnp.sum(t) / denom
    centered = jnp.where(mask, t - mu, 0.0)
    var = jnp.sum(centered * centered) / denom
    out = centered * lax.rsqrt(var + EPS) * lnw + lnb
    out = jnp.where(out > 0, out, a * out)
    return jnp.where(mask, out, 0.0)


def _tc_dense0_body(xp_ref, aggp_ref, cntp_ref, Wl_ref, bl_ref, Wr_ref,
                    lnw_ref, lnb_ref, a_ref, Wskip_ref, h1_ref):
    xp = xp_ref[...]
    h0 = _dense_layer(aggp_ref[...], cntp_ref[...], xp, Wl_ref[...],
                      bl_ref[...], Wr_ref[...], lnw_ref[...], lnb_ref[...],
                      a_ref[0, 0])
    h1 = jnp.dot(xp, Wskip_ref[...].T, preferred_element_type=jnp.float32) + h0
    h1_ref[...] = jnp.where(_row_mask(), h1, 0.0)


def _tc_dense1_body(h1_ref, aggp_ref, cntp_ref, Wl_ref, bl_ref, Wr_ref,
                    lnw_ref, lnb_ref, a_ref, out_ref):
    out_ref[...] = _dense_layer(aggp_ref[...], cntp_ref[...], h1_ref[...],
                                Wl_ref[...], bl_ref[...], Wr_ref[...],
                                lnw_ref[...], lnb_ref[...], a_ref[0, 0])


_tc_dense0 = pl.pallas_call(
    _tc_dense0_body,
    out_shape=jax.ShapeDtypeStruct((NPAD, D), jnp.float32),
)

_tc_dense1 = pl.pallas_call(
    _tc_dense1_body,
    out_shape=jax.ShapeDtypeStruct((NPAD, D), jnp.float32),
)


def kernel(x, edge_index, Wl0, bl0, Wr0, lnw0, lnb0, a0, Wskip,
           Wl1, bl1, Wr1, lnw1, lnb1, a1):
    pad = EPAD - E
    srcp = jnp.concatenate(
        [edge_index[0], jnp.zeros((pad,), jnp.int32)]).reshape(NW, NCH, CHUNK)
    dstp = jnp.concatenate(
        [edge_index[1], jnp.full((pad,), N, jnp.int32)]).reshape(NW, NCH, CHUNK)
    xp = jnp.pad(x, ((0, NPAD - N), (0, 0)))
    znd = jnp.zeros((NPAD, D), jnp.float32)
    bl0r = bl0.reshape(1, D)
    bl1r = bl1.reshape(1, D)
    lnw0r = lnw0.reshape(1, D)
    lnb0r = lnb0.reshape(1, D)
    lnw1r = lnw1.reshape(1, D)
    lnb1r = lnb1.reshape(1, D)
    a0r = a0.reshape(1, 1)
    a1r = a1.reshape(1, 1)

    (cntp,) = _sc_cnt(dstp)
    (aggp0,) = _sc_agg(xp, srcp, dstp, znd)
    h1p = _tc_dense0(xp, aggp0, cntp, Wl0, bl0r, Wr0, lnw0r, lnb0r, a0r, Wskip)
    (aggp1,) = _sc_agg(h1p, srcp, dstp, znd)
    outp = _tc_dense1(h1p, aggp1, cntp, Wl1, bl1r, Wr1, lnw1r, lnb1r, a1r)
    return outp[:N]
